# baseline (speedup 1.0000x reference)
import functools

import jax
import jax.numpy as jnp
import numpy as np
from jax import lax
from jax.experimental import pallas as pl
from jax.experimental.pallas import tpu as pltpu
from jax.experimental.pallas import tpu_sc as plsc

N = 10000
E = 320000
DI = 128
DH = 128
DO = 64

NP = 10240
NC = 2
NS = 16
NW = NC * NS
B = 128
G = 1
NB = 80
NB2 = 160
NBB = NB2 // G
EP = NW * NB * B
RPS = NP // NS


@functools.lru_cache(maxsize=None)
def _mesh():
    return plsc.VectorSubcoreMesh(
        core_axis_name="c", subcore_axis_name="s", num_cores=NC, num_subcores=NS
    )


def _deg_body(init_hbm, dst_hbm, out_hbm, dst_v, ones_v, acc, sem):
    c = lax.axis_index("c")
    s = lax.axis_index("s")
    wid = s * NC + c
    pltpu.sync_copy(dst_hbm.at[wid], dst_v)
    for t in range(B // 16):
        ones_v[pl.ds(t * 16, 16)] = jnp.ones((16,), jnp.float32)
    pltpu.sync_copy(init_hbm.at[c, pl.ds(s * RPS, RPS)], acc.at[pl.ds(s * RPS, RPS)])
    plsc.subcore_barrier()

    def body(j, carry):
        pltpu.sync_copy(ones_v, acc.at[dst_v.at[j]], add=True)
        return carry

    lax.fori_loop(0, NB, body, 0)
    plsc.subcore_barrier()
    pltpu.sync_copy(acc.at[pl.ds(s * RPS, RPS)], out_hbm.at[c, pl.ds(s * RPS, RPS)])


@functools.lru_cache(maxsize=None)
def _deg_kernel():
    return pl.kernel(
        _deg_body,
        out_type=jax.ShapeDtypeStruct((NC, NP), jnp.float32),
        mesh=_mesh(),
        scratch_types=[
            pltpu.VMEM((NB, B), jnp.int32),
            pltpu.VMEM((B,), jnp.float32),
            pltpu.VMEM_SHARED((NP,), jnp.float32),
            pltpu.SemaphoreType.DMA,
        ],
        compiler_params=pltpu.CompilerParams(use_tc_tiling_on_sc=False),
    )


def _agg_body(ys_hbm, src0_hbm, src1_hbm, dst_hbm, out_hbm,
              src_v, dst_v, rows0_v, rows1_v, acc, sg0, sg1, ss0, ss1):
    c = lax.axis_index("c")
    s = lax.axis_index("s")

    @pl.when(c == 0)
    def _():
        pltpu.sync_copy(src0_hbm.at[s], src_v)

    @pl.when(c == 1)
    def _():
        pltpu.sync_copy(src1_hbm.at[s], src_v)

    pltpu.sync_copy(dst_hbm.at[s], dst_v)

    pltpu.sync_copy(ys_hbm.at[pl.ds(c * NP + s * RPS, RPS)],
                    acc.at[pl.ds(s * RPS, RPS)])
    plsc.subcore_barrier()

    pltpu.async_copy(ys_hbm.at[src_v.at[0]], rows0_v, sg0)
    pltpu.async_copy(ys_hbm.at[src_v.at[1]], rows1_v, sg1)

    def body(jj, carry):
        j = jj * 2
        pltpu.make_async_copy(ys_hbm.at[src_v.at[0]], rows0_v, sg0).wait()
        ds0 = pltpu.async_copy(rows0_v, acc.at[dst_v.at[j]], ss0, add=True)
        pltpu.make_async_copy(ys_hbm.at[src_v.at[0]], rows1_v, sg1).wait()
        ds1 = pltpu.async_copy(rows1_v, acc.at[dst_v.at[j + 1]], ss1, add=True)
        ds0.wait()

        @pl.when(jj + 1 < NB2 // 2)
        def _():
            pltpu.async_copy(ys_hbm.at[src_v.at[j + 2]], rows0_v, sg0)

        ds1.wait()

        @pl.when(jj + 1 < NB2 // 2)
        def _():
            pltpu.async_copy(ys_hbm.at[src_v.at[j + 3]], rows1_v, sg1)

        return carry

    lax.fori_loop(0, NB2 // 2, body, 0)
    plsc.subcore_barrier()
    pltpu.sync_copy(acc.at[pl.ds(s * RPS, RPS)], out_hbm.at[c, pl.ds(s * RPS, RPS)])


@functools.lru_cache(maxsize=None)
def _make_agg(d):
    return pl.kernel(
        _agg_body,
        out_type=jax.ShapeDtypeStruct((NC, NP, d), jnp.float32),
        mesh=_mesh(),
        scratch_types=[
            pltpu.VMEM((NB2, B), jnp.int32),
            pltpu.VMEM((NB2, B), jnp.int32),
            pltpu.VMEM((B, d), jnp.float32),
            pltpu.VMEM((B, d), jnp.float32),
            pltpu.VMEM_SHARED((NP, d), jnp.float32),
            pltpu.SemaphoreType.DMA,
            pltpu.SemaphoreType.DMA,
            pltpu.SemaphoreType.DMA,
            pltpu.SemaphoreType.DMA,
        ],
        compiler_params=pltpu.CompilerParams(use_tc_tiling_on_sc=False),
    )


def _tc_mm_body(x_ref, w_ref, xw_ref):
    xw_ref[...] = jnp.dot(x_ref[...], w_ref[...],
                          preferred_element_type=jnp.float32)


def _tc_pre_body(xw_ref, dp_ref, ys_ref, dis_ref):
    deg = dp_ref[0] + dp_ref[1]
    dis = jnp.where(deg > 0.0, lax.rsqrt(deg), 0.0)
    ys = xw_ref[...] * dis
    h = DH // 2
    ys_ref[...] = jnp.stack([ys[:, :h], ys[:, h:]])
    dis_ref[...] = dis


def _tc_mid_body(p_ref, dis_ref, b1_ref, w_ref, ys2_ref):
    dis = dis_ref[...]
    agg = jnp.concatenate([p_ref[0], p_ref[1]], axis=1)
    h = jnp.maximum(dis * agg + b1_ref[...], 0.0)
    hw = jnp.dot(h, w_ref[...], preferred_element_type=jnp.float32)
    ys2 = hw * dis
    ho = DO // 2
    ys2_ref[...] = jnp.stack([ys2[:, :ho], ys2[:, ho:]])


def _tc_post_body(q_ref, dis_ref, b2_ref, out_ref):
    agg = jnp.concatenate([q_ref[0], q_ref[1]], axis=1)
    out_ref[...] = dis_ref[...] * agg + b2_ref[...]


def _tc_mm(x_pad, W1):
    return pl.pallas_call(
        _tc_mm_body,
        in_specs=[
            pl.BlockSpec((NP, DI), lambda: (0, 0)),
            pl.BlockSpec((DI, DH), lambda: (0, 0)),
        ],
        out_specs=pl.BlockSpec((NP, DH), lambda: (0, 0)),
        out_shape=jax.ShapeDtypeStruct((NP, DH), jnp.float32),
    )(x_pad, W1)


def _tc_pre(xw, degp):
    return pl.pallas_call(
        _tc_pre_body,
        in_specs=[
            pl.BlockSpec((NP, DH), lambda: (0, 0)),
            pl.BlockSpec((NC, NP, 1), lambda: (0, 0, 0)),
        ],
        out_specs=[
            pl.BlockSpec((NC, NP, DH // 2), lambda: (0, 0, 0)),
            pl.BlockSpec((NP, 1), lambda: (0, 0)),
        ],
        out_shape=[
            jax.ShapeDtypeStruct((NC, NP, DH // 2), jnp.float32),
            jax.ShapeDtypeStruct((NP, 1), jnp.float32),
        ],
    )(xw, degp)


def _tc_mid(p, dis, b1, W2):
    return pl.pallas_call(
        _tc_mid_body,
        in_specs=[
            pl.BlockSpec((NC, NP, DH // 2), lambda: (0, 0, 0)),
            pl.BlockSpec((NP, 1), lambda: (0, 0)),
            pl.BlockSpec((1, DH), lambda: (0, 0)),
            pl.BlockSpec((DH, DO), lambda: (0, 0)),
        ],
        out_specs=pl.BlockSpec((NC, NP, DO // 2), lambda: (0, 0, 0)),
        out_shape=jax.ShapeDtypeStruct((NC, NP, DO // 2), jnp.float32),
    )(p, dis, b1, W2)


def _tc_post(q, dis, b2):
    R = 1000
    return pl.pallas_call(
        _tc_post_body,
        grid=(N // R,),
        in_specs=[
            pl.BlockSpec((NC, R, DO // 2), lambda i: (0, i, 0)),
            pl.BlockSpec((R, 1), lambda i: (i, 0)),
            pl.BlockSpec((1, DO), lambda i: (0, 0)),
        ],
        out_specs=pl.BlockSpec((R, DO), lambda i: (i, 0)),
        out_shape=jax.ShapeDtypeStruct((N, DO), jnp.float32),
    )(q, dis, b2)


_PAD_SRC = np.asarray((np.arange(EP - E) * 97) % N, np.int32)
_PAD_DST = np.asarray(N + np.arange(EP - E) % (NP - N), np.int32)


def kernel(x, edge_index, W1, b1, W2, b2):
    src = edge_index[0].astype(jnp.int32)
    dst = edge_index[1].astype(jnp.int32)
    src_p = jnp.concatenate([src, jnp.asarray(_PAD_SRC)])
    dst_p = jnp.concatenate([dst, jnp.asarray(_PAD_DST)])
    dst_deg = dst_p.reshape(NW, NB, B)
    src_r0 = src_p.reshape(NS, NB2, B)
    src_r1 = src_r0 + NP
    dst_r = dst_p.reshape(NS, NB2, B)

    x_pad = jnp.pad(x, ((0, NP - N), (0, 0)))
    deg_init = jnp.concatenate(
        [jnp.ones((1, NP), jnp.float32), jnp.zeros((1, NP), jnp.float32)]
    )

    degp = _deg_kernel()(deg_init, dst_deg).reshape(NC, NP, 1)
    xw = _tc_mm(x_pad, W1)
    ys, dis = _tc_pre(xw, degp)
    ys_cat = ys.reshape(NC * NP, DH // 2)
    p = _make_agg(DH // 2)(ys_cat, src_r0, src_r1, dst_r)
    ys2 = _tc_mid(p, dis, b1.reshape(1, DH), W2)
    ys2_cat = ys2.reshape(NC * NP, DO // 2)
    q = _make_agg(DO // 2)(ys2_cat, src_r0, src_r1, dst_r)
    out = _tc_post(q, dis, b2.reshape(1, DO))
    return out


# speedup vs baseline: 1.1387x; 1.1387x over previous
import functools

import jax
import jax.numpy as jnp
import numpy as np
from jax import lax
from jax.experimental import pallas as pl
from jax.experimental.pallas import tpu as pltpu
from jax.experimental.pallas import tpu_sc as plsc

N = 10000
E = 320000
DI = 128
DH = 128
DO = 64

NP = 10240
NC = 2
NS = 16
NW = NC * NS
B = 128
G = 1
NB = 80
NB2 = 160
NBB = NB2 // G
EP = NW * NB * B
RPS = NP // NS


@functools.lru_cache(maxsize=None)
def _mesh():
    return plsc.VectorSubcoreMesh(
        core_axis_name="c", subcore_axis_name="s", num_cores=NC, num_subcores=NS
    )


def _deg_body(init_hbm, dst_hbm, out_hbm, dst_v, ones_v, acc, sem):
    c = lax.axis_index("c")
    s = lax.axis_index("s")
    wid = s * NC + c
    pltpu.sync_copy(dst_hbm.at[wid], dst_v)
    for t in range(B // 16):
        ones_v[pl.ds(t * 16, 16)] = jnp.ones((16,), jnp.float32)
    pltpu.sync_copy(init_hbm.at[c, pl.ds(s * RPS, RPS)], acc.at[pl.ds(s * RPS, RPS)])
    plsc.subcore_barrier()

    def body(j, carry):
        pltpu.sync_copy(ones_v, acc.at[dst_v.at[j]], add=True)
        return carry

    lax.fori_loop(0, NB, body, 0)
    plsc.subcore_barrier()
    pltpu.sync_copy(acc.at[pl.ds(s * RPS, RPS)], out_hbm.at[c, pl.ds(s * RPS, RPS)])


@functools.lru_cache(maxsize=None)
def _deg_kernel():
    return pl.kernel(
        _deg_body,
        out_type=jax.ShapeDtypeStruct((NC, NP), jnp.float32),
        mesh=_mesh(),
        scratch_types=[
            pltpu.VMEM((NB, B), jnp.int32),
            pltpu.VMEM((B,), jnp.float32),
            pltpu.VMEM_SHARED((NP,), jnp.float32),
            pltpu.SemaphoreType.DMA,
        ],
        compiler_params=pltpu.CompilerParams(use_tc_tiling_on_sc=False),
    )


def _agg_body(ys_hbm, src0_hbm, src1_hbm, dst_hbm, out_hbm,
              src_v, dst_v, rows0_v, rows1_v, acc, sg0, sg1, ss0, ss1):
    c = lax.axis_index("c")
    s = lax.axis_index("s")

    @pl.when(c == 0)
    def _():
        pltpu.sync_copy(src0_hbm.at[s], src_v)

    @pl.when(c == 1)
    def _():
        pltpu.sync_copy(src1_hbm.at[s], src_v)

    pltpu.sync_copy(dst_hbm.at[s], dst_v)

    pltpu.sync_copy(ys_hbm.at[pl.ds(c * NP + s * RPS, RPS)],
                    acc.at[pl.ds(s * RPS, RPS)])
    plsc.subcore_barrier()

    pltpu.async_copy(ys_hbm.at[src_v.at[0]], rows0_v, sg0)

    def body(jj, carry):
        j = jj * 2
        d1 = pltpu.async_copy(ys_hbm.at[src_v.at[j + 1]], rows1_v, sg1)
        pltpu.make_async_copy(ys_hbm.at[src_v.at[0]], rows0_v, sg0).wait()
        pltpu.sync_copy(rows0_v, acc.at[dst_v.at[j]], add=True)

        @pl.when(jj + 1 < NB2 // 2)
        def _():
            pltpu.async_copy(ys_hbm.at[src_v.at[j + 2]], rows0_v, sg0)

        d1.wait()
        pltpu.sync_copy(rows1_v, acc.at[dst_v.at[j + 1]], add=True)
        return carry

    lax.fori_loop(0, NB2 // 2, body, 0)
    plsc.subcore_barrier()
    pltpu.sync_copy(acc.at[pl.ds(s * RPS, RPS)], out_hbm.at[c, pl.ds(s * RPS, RPS)])


@functools.lru_cache(maxsize=None)
def _make_agg(d):
    return pl.kernel(
        _agg_body,
        out_type=jax.ShapeDtypeStruct((NC, NP, d), jnp.float32),
        mesh=_mesh(),
        scratch_types=[
            pltpu.VMEM((NB2, B), jnp.int32),
            pltpu.VMEM((NB2, B), jnp.int32),
            pltpu.VMEM((B, d), jnp.float32),
            pltpu.VMEM((B, d), jnp.float32),
            pltpu.VMEM_SHARED((NP, d), jnp.float32),
            pltpu.SemaphoreType.DMA,
            pltpu.SemaphoreType.DMA,
            pltpu.SemaphoreType.DMA,
            pltpu.SemaphoreType.DMA,
        ],
        compiler_params=pltpu.CompilerParams(use_tc_tiling_on_sc=False),
    )


def _tc_mm_body(x_ref, w_ref, xw_ref):
    xw_ref[...] = jnp.dot(x_ref[...], w_ref[...],
                          preferred_element_type=jnp.float32)


def _tc_pre_body(xw_ref, dp_ref, ys_ref, dis_ref):
    deg = dp_ref[0] + dp_ref[1]
    dis = jnp.where(deg > 0.0, lax.rsqrt(deg), 0.0)
    ys = xw_ref[...] * dis
    h = DH // 2
    ys_ref[...] = jnp.stack([ys[:, :h], ys[:, h:]])
    dis_ref[...] = dis


def _tc_mid_body(p_ref, dis_ref, b1_ref, w_ref, ys2_ref):
    dis = dis_ref[...]
    agg = jnp.concatenate([p_ref[0], p_ref[1]], axis=1)
    h = jnp.maximum(dis * agg + b1_ref[...], 0.0)
    hw = jnp.dot(h, w_ref[...], preferred_element_type=jnp.float32)
    ys2 = hw * dis
    ho = DO // 2
    ys2_ref[...] = jnp.stack([ys2[:, :ho], ys2[:, ho:]])


def _tc_post_body(q_ref, dis_ref, b2_ref, out_ref):
    agg = jnp.concatenate([q_ref[0], q_ref[1]], axis=1)
    out_ref[...] = dis_ref[...] * agg + b2_ref[...]


def _tc_mm(x_pad, W1):
    return pl.pallas_call(
        _tc_mm_body,
        in_specs=[
            pl.BlockSpec((NP, DI), lambda: (0, 0)),
            pl.BlockSpec((DI, DH), lambda: (0, 0)),
        ],
        out_specs=pl.BlockSpec((NP, DH), lambda: (0, 0)),
        out_shape=jax.ShapeDtypeStruct((NP, DH), jnp.float32),
    )(x_pad, W1)


def _tc_pre(xw, degp):
    return pl.pallas_call(
        _tc_pre_body,
        in_specs=[
            pl.BlockSpec((NP, DH), lambda: (0, 0)),
            pl.BlockSpec((NC, NP, 1), lambda: (0, 0, 0)),
        ],
        out_specs=[
            pl.BlockSpec((NC, NP, DH // 2), lambda: (0, 0, 0)),
            pl.BlockSpec((NP, 1), lambda: (0, 0)),
        ],
        out_shape=[
            jax.ShapeDtypeStruct((NC, NP, DH // 2), jnp.float32),
            jax.ShapeDtypeStruct((NP, 1), jnp.float32),
        ],
    )(xw, degp)


def _tc_mid(p, dis, b1, W2):
    return pl.pallas_call(
        _tc_mid_body,
        in_specs=[
            pl.BlockSpec((NC, NP, DH // 2), lambda: (0, 0, 0)),
            pl.BlockSpec((NP, 1), lambda: (0, 0)),
            pl.BlockSpec((1, DH), lambda: (0, 0)),
            pl.BlockSpec((DH, DO), lambda: (0, 0)),
        ],
        out_specs=pl.BlockSpec((NC, NP, DO // 2), lambda: (0, 0, 0)),
        out_shape=jax.ShapeDtypeStruct((NC, NP, DO // 2), jnp.float32),
    )(p, dis, b1, W2)


def _tc_post(q, dis, b2):
    R = 1000
    return pl.pallas_call(
        _tc_post_body,
        grid=(N // R,),
        in_specs=[
            pl.BlockSpec((NC, R, DO // 2), lambda i: (0, i, 0)),
            pl.BlockSpec((R, 1), lambda i: (i, 0)),
            pl.BlockSpec((1, DO), lambda i: (0, 0)),
        ],
        out_specs=pl.BlockSpec((R, DO), lambda i: (i, 0)),
        out_shape=jax.ShapeDtypeStruct((N, DO), jnp.float32),
    )(q, dis, b2)


_PAD_SRC = np.asarray((np.arange(EP - E) * 97) % N, np.int32)
_PAD_DST = np.asarray(N + np.arange(EP - E) % (NP - N), np.int32)


def kernel(x, edge_index, W1, b1, W2, b2):
    src = edge_index[0].astype(jnp.int32)
    dst = edge_index[1].astype(jnp.int32)
    src_p = jnp.concatenate([src, jnp.asarray(_PAD_SRC)])
    dst_p = jnp.concatenate([dst, jnp.asarray(_PAD_DST)])
    dst_deg = dst_p.reshape(NW, NB, B)
    src_r0 = src_p.reshape(NS, NB2, B)
    src_r1 = src_r0 + NP
    dst_r = dst_p.reshape(NS, NB2, B)

    x_pad = jnp.pad(x, ((0, NP - N), (0, 0)))
    deg_init = jnp.concatenate(
        [jnp.ones((1, NP), jnp.float32), jnp.zeros((1, NP), jnp.float32)]
    )

    degp = _deg_kernel()(deg_init, dst_deg).reshape(NC, NP, 1)
    xw = _tc_mm(x_pad, W1)
    ys, dis = _tc_pre(xw, degp)
    ys_cat = ys.reshape(NC * NP, DH // 2)
    p = _make_agg(DH // 2)(ys_cat, src_r0, src_r1, dst_r)
    ys2 = _tc_mid(p, dis, b1.reshape(1, DH), W2)
    ys2_cat = ys2.reshape(NC * NP, DO // 2)
    q = _make_agg(DO // 2)(ys2_cat, src_r0, src_r1, dst_r)
    out = _tc_post(q, dis, b2.reshape(1, DO))
    return out


# speedup vs baseline: 1.3394x; 1.1762x over previous
import functools

import jax
import jax.numpy as jnp
import numpy as np
from jax import lax
from jax.experimental import pallas as pl
from jax.experimental.pallas import tpu as pltpu
from jax.experimental.pallas import tpu_sc as plsc

N = 10000
E = 320000
DI = 128
DH = 128
DO = 64

NP = 10240
NC = 2
NS = 16
NW = NC * NS
B = 256
NB = 40
NB2 = 80
EP = NW * NB * B
RPS = NP // NS


@functools.lru_cache(maxsize=None)
def _mesh():
    return plsc.VectorSubcoreMesh(
        core_axis_name="c", subcore_axis_name="s", num_cores=NC, num_subcores=NS
    )


def _deg_body(init_hbm, dst_hbm, out_hbm, dst_v, ones_v, acc, sem):
    c = lax.axis_index("c")
    s = lax.axis_index("s")
    wid = s * NC + c
    pltpu.sync_copy(dst_hbm.at[wid], dst_v)
    for t in range(B // 16):
        ones_v[pl.ds(t * 16, 16)] = jnp.ones((16,), jnp.float32)
    pltpu.sync_copy(init_hbm.at[c, pl.ds(s * RPS, RPS)], acc.at[pl.ds(s * RPS, RPS)])
    plsc.subcore_barrier()

    def body(j, carry):
        pltpu.sync_copy(ones_v, acc.at[dst_v.at[j]], add=True)
        return carry

    lax.fori_loop(0, NB, body, 0)
    plsc.subcore_barrier()
    pltpu.sync_copy(acc.at[pl.ds(s * RPS, RPS)], out_hbm.at[c, pl.ds(s * RPS, RPS)])


@functools.lru_cache(maxsize=None)
def _deg_kernel():
    return pl.kernel(
        _deg_body,
        out_type=jax.ShapeDtypeStruct((NC, NP), jnp.float32),
        mesh=_mesh(),
        scratch_types=[
            pltpu.VMEM((NB, B), jnp.int32),
            pltpu.VMEM((B,), jnp.float32),
            pltpu.VMEM_SHARED((NP,), jnp.float32),
            pltpu.SemaphoreType.DMA,
        ],
        compiler_params=pltpu.CompilerParams(use_tc_tiling_on_sc=False),
    )


def _agg_body(ys_hbm, src0_hbm, src1_hbm, dst_hbm, out_hbm,
              src_v, dst_v, rows0_v, rows1_v, acc, sg0, sg1):
    c = lax.axis_index("c")
    s = lax.axis_index("s")

    @pl.when(c == 0)
    def _():
        pltpu.sync_copy(src0_hbm.at[s], src_v)

    @pl.when(c == 1)
    def _():
        pltpu.sync_copy(src1_hbm.at[s], src_v)

    pltpu.sync_copy(dst_hbm.at[s], dst_v)

    pltpu.sync_copy(ys_hbm.at[pl.ds(c * NP + s * RPS, RPS)],
                    acc.at[pl.ds(s * RPS, RPS)])
    plsc.subcore_barrier()

    pltpu.async_copy(ys_hbm.at[src_v.at[0]], rows0_v, sg0)

    def body(jj, carry):
        j = jj * 2
        d1 = pltpu.async_copy(ys_hbm.at[src_v.at[j + 1]], rows1_v, sg1)
        pltpu.make_async_copy(ys_hbm.at[src_v.at[0]], rows0_v, sg0).wait()
        pltpu.sync_copy(rows0_v, acc.at[dst_v.at[j]], add=True)

        @pl.when(jj + 1 < NB2 // 2)
        def _():
            pltpu.async_copy(ys_hbm.at[src_v.at[j + 2]], rows0_v, sg0)

        d1.wait()
        pltpu.sync_copy(rows1_v, acc.at[dst_v.at[j + 1]], add=True)
        return carry

    lax.fori_loop(0, NB2 // 2, body, 0)
    plsc.subcore_barrier()
    pltpu.sync_copy(acc.at[pl.ds(s * RPS, RPS)], out_hbm.at[c, pl.ds(s * RPS, RPS)])


@functools.lru_cache(maxsize=None)
def _make_agg(d):
    return pl.kernel(
        _agg_body,
        out_type=jax.ShapeDtypeStruct((NC, NP, d), jnp.float32),
        mesh=_mesh(),
        scratch_types=[
            pltpu.VMEM((NB2, B), jnp.int32),
            pltpu.VMEM((NB2, B), jnp.int32),
            pltpu.VMEM((B, d), jnp.float32),
            pltpu.VMEM((B, d), jnp.float32),
            pltpu.VMEM_SHARED((NP, d), jnp.float32),
            pltpu.SemaphoreType.DMA,
            pltpu.SemaphoreType.DMA,
        ],
        compiler_params=pltpu.CompilerParams(use_tc_tiling_on_sc=False),
    )


def _tc_mm_body(x_ref, w0_ref, w1_ref, xw_ref):
    x = x_ref[...]
    xw_ref[0] = jnp.dot(x, w0_ref[...], preferred_element_type=jnp.float32)
    xw_ref[1] = jnp.dot(x, w1_ref[...], preferred_element_type=jnp.float32)


def _tc_pre_body(xw_ref, dp_ref, ys_ref, dis_ref):
    deg = dp_ref[0] + dp_ref[1]
    dis_c = jnp.where(deg > 0.0, lax.rsqrt(deg), 0.0).reshape(NP, 1)
    ys_ref[0] = xw_ref[0] * dis_c
    ys_ref[1] = xw_ref[1] * dis_c
    dis_ref[...] = dis_c


def _tc_mid_body(p_ref, dis_ref, b10_ref, b11_ref,
                 w00_ref, w01_ref, w10_ref, w11_ref, ys2_ref):
    dis = dis_ref[...]
    h0 = jnp.maximum(dis * p_ref[0] + b10_ref[...], 0.0)
    h1 = jnp.maximum(dis * p_ref[1] + b11_ref[...], 0.0)
    hw0 = (jnp.dot(h0, w00_ref[...], preferred_element_type=jnp.float32)
           + jnp.dot(h1, w10_ref[...], preferred_element_type=jnp.float32))
    hw1 = (jnp.dot(h0, w01_ref[...], preferred_element_type=jnp.float32)
           + jnp.dot(h1, w11_ref[...], preferred_element_type=jnp.float32))
    ys2_ref[0] = hw0 * dis
    ys2_ref[1] = hw1 * dis


def _tc_post_body(q_ref, dis_ref, b2_ref, out_ref):
    agg = jnp.concatenate([q_ref[0], q_ref[1]], axis=1)
    out_ref[...] = dis_ref[...] * agg + b2_ref[...]


def _tc_mm(x_pad, W1a, W1b):
    return pl.pallas_call(
        _tc_mm_body,
        in_specs=[
            pl.BlockSpec((NP, DI), lambda: (0, 0)),
            pl.BlockSpec((DI, DH // 2), lambda: (0, 0)),
            pl.BlockSpec((DI, DH // 2), lambda: (0, 0)),
        ],
        out_specs=pl.BlockSpec((NC, NP, DH // 2), lambda: (0, 0, 0)),
        out_shape=jax.ShapeDtypeStruct((NC, NP, DH // 2), jnp.float32),
    )(x_pad, W1a, W1b)


def _tc_pre(xw, degp):
    return pl.pallas_call(
        _tc_pre_body,
        in_specs=[
            pl.BlockSpec((NC, NP, DH // 2), lambda: (0, 0, 0)),
            pl.BlockSpec((NC, NP), lambda: (0, 0)),
        ],
        out_specs=[
            pl.BlockSpec((NC, NP, DH // 2), lambda: (0, 0, 0)),
            pl.BlockSpec((NP, 1), lambda: (0, 0)),
        ],
        out_shape=[
            jax.ShapeDtypeStruct((NC, NP, DH // 2), jnp.float32),
            jax.ShapeDtypeStruct((NP, 1), jnp.float32),
        ],
    )(xw, degp)


def _tc_mid(p, dis, b10, b11, W2s):
    return pl.pallas_call(
        _tc_mid_body,
        in_specs=[
            pl.BlockSpec((NC, NP, DH // 2), lambda: (0, 0, 0)),
            pl.BlockSpec((NP, 1), lambda: (0, 0)),
            pl.BlockSpec((1, DH // 2), lambda: (0, 0)),
            pl.BlockSpec((1, DH // 2), lambda: (0, 0)),
            pl.BlockSpec((DH // 2, DO // 2), lambda: (0, 0)),
            pl.BlockSpec((DH // 2, DO // 2), lambda: (0, 0)),
            pl.BlockSpec((DH // 2, DO // 2), lambda: (0, 0)),
            pl.BlockSpec((DH // 2, DO // 2), lambda: (0, 0)),
        ],
        out_specs=pl.BlockSpec((NC, NP, DO // 2), lambda: (0, 0, 0)),
        out_shape=jax.ShapeDtypeStruct((NC, NP, DO // 2), jnp.float32),
    )(p, dis, b10, b11, *W2s)


def _tc_post(q, dis, b2):
    R = 1000
    return pl.pallas_call(
        _tc_post_body,
        grid=(N // R,),
        in_specs=[
            pl.BlockSpec((NC, R, DO // 2), lambda i: (0, i, 0)),
            pl.BlockSpec((R, 1), lambda i: (i, 0)),
            pl.BlockSpec((1, DO), lambda i: (0, 0)),
        ],
        out_specs=pl.BlockSpec((R, DO), lambda i: (i, 0)),
        out_shape=jax.ShapeDtypeStruct((N, DO), jnp.float32),
    )(q, dis, b2)


_PAD_EDGES = np.stack([
    np.asarray((np.arange(EP - E) * 97) % N, np.int32),
    np.asarray(N + np.arange(EP - E) % (NP - N), np.int32),
])


def kernel(x, edge_index, W1, b1, W2, b2):
    ei = jnp.concatenate([edge_index.astype(jnp.int32),
                          jnp.asarray(_PAD_EDGES)], axis=1)
    dst_deg = ei[1].reshape(NW, NB, B)
    src_r0 = ei[0].reshape(NS, NB2, B)
    src_r1 = src_r0 + NP
    dst_r = ei[1].reshape(NS, NB2, B)

    x_pad = jnp.pad(x, ((0, NP - N), (0, 0)))
    deg_init = jnp.concatenate(
        [jnp.ones((1, NP), jnp.float32), jnp.zeros((1, NP), jnp.float32)]
    )
    h = DH // 2
    ho = DO // 2

    degp = _deg_kernel()(deg_init, dst_deg)
    xw = _tc_mm(x_pad, W1[:, :h], W1[:, h:])
    ys, dis = _tc_pre(xw, degp)
    ys_cat = ys.reshape(NC * NP, h)
    p = _make_agg(h)(ys_cat, src_r0, src_r1, dst_r)
    W2s = (W2[:h, :ho], W2[:h, ho:], W2[h:, :ho], W2[h:, ho:])
    ys2 = _tc_mid(p, dis, b1.reshape(1, DH)[:, :h], b1.reshape(1, DH)[:, h:], W2s)
    ys2_cat = ys2.reshape(NC * NP, ho)
    q = _make_agg(ho)(ys2_cat, src_r0, src_r1, dst_r)
    out = _tc_post(q, dis, b2.reshape(1, DO))
    return out


# speedup vs baseline: 1.4140x; 1.0557x over previous
import functools

import jax
import jax.numpy as jnp
import numpy as np
from jax import lax
from jax.experimental import pallas as pl
from jax.experimental.pallas import tpu as pltpu
from jax.experimental.pallas import tpu_sc as plsc

N = 10000
E = 320000
DI = 128
DH = 128
DO = 64

NP = 10240
NC = 2
NS = 16
NW = NC * NS
B = 256
NB = 40
NB2 = 80
EP = NW * NB * B
RPS = NP // NS


@functools.lru_cache(maxsize=None)
def _mesh():
    return plsc.VectorSubcoreMesh(
        core_axis_name="c", subcore_axis_name="s", num_cores=NC, num_subcores=NS
    )


def _deg_body(init_hbm, dst_hbm, out_hbm, dst_v, ones_v, acc, sem):
    c = lax.axis_index("c")
    s = lax.axis_index("s")
    wid = s * NC + c
    pltpu.sync_copy(dst_hbm.at[wid], dst_v)
    for t in range(B // 16):
        ones_v[pl.ds(t * 16, 16)] = jnp.ones((16,), jnp.float32)
    pltpu.sync_copy(init_hbm.at[c, pl.ds(s * RPS, RPS)], acc.at[pl.ds(s * RPS, RPS)])
    plsc.subcore_barrier()

    def body(j, carry):
        pltpu.sync_copy(ones_v, acc.at[dst_v.at[j]], add=True)
        return carry

    lax.fori_loop(0, NB, body, 0)
    plsc.subcore_barrier()
    pltpu.sync_copy(acc.at[pl.ds(s * RPS, RPS)], out_hbm.at[c, pl.ds(s * RPS, RPS)])


@functools.lru_cache(maxsize=None)
def _deg_kernel():
    return pl.kernel(
        _deg_body,
        out_type=jax.ShapeDtypeStruct((NC, NP), jnp.float32),
        mesh=_mesh(),
        scratch_types=[
            pltpu.VMEM((NB, B), jnp.int32),
            pltpu.VMEM((B,), jnp.float32),
            pltpu.VMEM_SHARED((NP,), jnp.float32),
            pltpu.SemaphoreType.DMA,
        ],
        compiler_params=pltpu.CompilerParams(use_tc_tiling_on_sc=False),
    )


def _agg_body(ys_hbm, src0_hbm, dst_hbm, out_hbm,
              src_v, dst_v, rows0_v, rows1_v, acc, sg0, sg1):
    c = lax.axis_index("c")
    s = lax.axis_index("s")

    tbl = ys_hbm.at[pl.ds(c * NP, NP)]
    pltpu.sync_copy(src0_hbm.at[s], src_v)
    pltpu.sync_copy(dst_hbm.at[s], dst_v)

    pltpu.sync_copy(ys_hbm.at[pl.ds(c * NP + s * RPS, RPS)],
                    acc.at[pl.ds(s * RPS, RPS)])
    plsc.subcore_barrier()

    pltpu.async_copy(tbl.at[src_v.at[0]], rows0_v, sg0)

    def body(jj, carry):
        j = jj * 2
        d1 = pltpu.async_copy(tbl.at[src_v.at[j + 1]], rows1_v, sg1)
        pltpu.make_async_copy(tbl.at[src_v.at[0]], rows0_v, sg0).wait()
        pltpu.sync_copy(rows0_v, acc.at[dst_v.at[j]], add=True)

        @pl.when(jj + 1 < NB2 // 2)
        def _():
            pltpu.async_copy(tbl.at[src_v.at[j + 2]], rows0_v, sg0)

        d1.wait()
        pltpu.sync_copy(rows1_v, acc.at[dst_v.at[j + 1]], add=True)
        return carry

    lax.fori_loop(0, NB2 // 2, body, 0)
    plsc.subcore_barrier()
    pltpu.sync_copy(acc.at[pl.ds(s * RPS, RPS)], out_hbm.at[c, pl.ds(s * RPS, RPS)])


@functools.lru_cache(maxsize=None)
def _make_agg(d):
    return pl.kernel(
        _agg_body,
        out_type=jax.ShapeDtypeStruct((NC, NP, d), jnp.float32),
        mesh=_mesh(),
        scratch_types=[
            pltpu.VMEM((NB2, B), jnp.int32),
            pltpu.VMEM((NB2, B), jnp.int32),
            pltpu.VMEM((B, d), jnp.float32),
            pltpu.VMEM((B, d), jnp.float32),
            pltpu.VMEM_SHARED((NP, d), jnp.float32),
            pltpu.SemaphoreType.DMA,
            pltpu.SemaphoreType.DMA,
        ],
        compiler_params=pltpu.CompilerParams(use_tc_tiling_on_sc=False),
    )


def _tc_mm_body(x_ref, w0_ref, w1_ref, xw_ref):
    x = x_ref[...]
    xw_ref[0] = jnp.dot(x, w0_ref[...], preferred_element_type=jnp.float32)
    xw_ref[1] = jnp.dot(x, w1_ref[...], preferred_element_type=jnp.float32)


def _tc_pre_body(xw_ref, dp_ref, ys_ref, dis_ref):
    deg = dp_ref[0] + dp_ref[1]
    dis_c = jnp.where(deg > 0.0, lax.rsqrt(deg), 0.0).reshape(NP, 1)
    ys_ref[0] = xw_ref[0] * dis_c
    ys_ref[1] = xw_ref[1] * dis_c
    dis_ref[...] = dis_c


def _tc_mid_body(p_ref, dis_ref, b10_ref, b11_ref,
                 w00_ref, w01_ref, w10_ref, w11_ref, ys2_ref):
    dis = dis_ref[...]
    h0 = jnp.maximum(dis * p_ref[0] + b10_ref[...], 0.0)
    h1 = jnp.maximum(dis * p_ref[1] + b11_ref[...], 0.0)
    hw0 = (jnp.dot(h0, w00_ref[...], preferred_element_type=jnp.float32)
           + jnp.dot(h1, w10_ref[...], preferred_element_type=jnp.float32))
    hw1 = (jnp.dot(h0, w01_ref[...], preferred_element_type=jnp.float32)
           + jnp.dot(h1, w11_ref[...], preferred_element_type=jnp.float32))
    ys2_ref[0] = hw0 * dis
    ys2_ref[1] = hw1 * dis


def _tc_post_body(q_ref, dis_ref, b2_ref, out_ref):
    agg = jnp.concatenate([q_ref[0], q_ref[1]], axis=1)
    out_ref[...] = dis_ref[...] * agg + b2_ref[...]


def _tc_mm(x_pad, W1a, W1b):
    return pl.pallas_call(
        _tc_mm_body,
        in_specs=[
            pl.BlockSpec((NP, DI), lambda: (0, 0)),
            pl.BlockSpec((DI, DH // 2), lambda: (0, 0)),
            pl.BlockSpec((DI, DH // 2), lambda: (0, 0)),
        ],
        out_specs=pl.BlockSpec((NC, NP, DH // 2), lambda: (0, 0, 0)),
        out_shape=jax.ShapeDtypeStruct((NC, NP, DH // 2), jnp.float32),
    )(x_pad, W1a, W1b)


def _tc_pre(xw, degp):
    return pl.pallas_call(
        _tc_pre_body,
        in_specs=[
            pl.BlockSpec((NC, NP, DH // 2), lambda: (0, 0, 0)),
            pl.BlockSpec((NC, NP), lambda: (0, 0)),
        ],
        out_specs=[
            pl.BlockSpec((NC, NP, DH // 2), lambda: (0, 0, 0)),
            pl.BlockSpec((NP, 1), lambda: (0, 0)),
        ],
        out_shape=[
            jax.ShapeDtypeStruct((NC, NP, DH // 2), jnp.float32),
            jax.ShapeDtypeStruct((NP, 1), jnp.float32),
        ],
    )(xw, degp)


def _tc_mid(p, dis, b10, b11, W2s):
    return pl.pallas_call(
        _tc_mid_body,
        in_specs=[
            pl.BlockSpec((NC, NP, DH // 2), lambda: (0, 0, 0)),
            pl.BlockSpec((NP, 1), lambda: (0, 0)),
            pl.BlockSpec((1, DH // 2), lambda: (0, 0)),
            pl.BlockSpec((1, DH // 2), lambda: (0, 0)),
            pl.BlockSpec((DH // 2, DO // 2), lambda: (0, 0)),
            pl.BlockSpec((DH // 2, DO // 2), lambda: (0, 0)),
            pl.BlockSpec((DH // 2, DO // 2), lambda: (0, 0)),
            pl.BlockSpec((DH // 2, DO // 2), lambda: (0, 0)),
        ],
        out_specs=pl.BlockSpec((NC, NP, DO // 2), lambda: (0, 0, 0)),
        out_shape=jax.ShapeDtypeStruct((NC, NP, DO // 2), jnp.float32),
    )(p, dis, b10, b11, *W2s)


def _tc_post(q, dis, b2):
    R = 1000
    return pl.pallas_call(
        _tc_post_body,
        grid=(N // R,),
        in_specs=[
            pl.BlockSpec((NC, R, DO // 2), lambda i: (0, i, 0)),
            pl.BlockSpec((R, 1), lambda i: (i, 0)),
            pl.BlockSpec((1, DO), lambda i: (0, 0)),
        ],
        out_specs=pl.BlockSpec((R, DO), lambda i: (i, 0)),
        out_shape=jax.ShapeDtypeStruct((N, DO), jnp.float32),
    )(q, dis, b2)


_PAD_EDGES = np.stack([
    np.asarray((np.arange(EP - E) * 97) % N, np.int32),
    np.asarray(N + np.arange(EP - E) % (NP - N), np.int32),
])


def kernel(x, edge_index, W1, b1, W2, b2):
    ei = jnp.concatenate([edge_index.astype(jnp.int32),
                          jnp.asarray(_PAD_EDGES)], axis=1)
    dst_deg = ei[1].reshape(NW, NB, B)
    src_r0 = ei[0].reshape(NS, NB2, B)
    dst_r = ei[1].reshape(NS, NB2, B)

    x_pad = jnp.pad(x, ((0, NP - N), (0, 0)))
    deg_init = jnp.concatenate(
        [jnp.ones((1, NP), jnp.float32), jnp.zeros((1, NP), jnp.float32)]
    )
    h = DH // 2
    ho = DO // 2

    degp = _deg_kernel()(deg_init, dst_deg)
    xw = _tc_mm(x_pad, W1[:, :h], W1[:, h:])
    ys, dis = _tc_pre(xw, degp)
    ys_cat = ys.reshape(NC * NP, h)
    p = _make_agg(h)(ys_cat, src_r0, dst_r)
    W2s = (W2[:h, :ho], W2[:h, ho:], W2[h:, :ho], W2[h:, ho:])
    ys2 = _tc_mid(p, dis, b1.reshape(1, DH)[:, :h], b1.reshape(1, DH)[:, h:], W2s)
    ys2_cat = ys2.reshape(NC * NP, ho)
    q = _make_agg(ho)(ys2_cat, src_r0, dst_r)
    out = _tc_post(q, dis, b2.reshape(1, DO))
    return out


# speedup vs baseline: 1.4211x; 1.0050x over previous
import functools

import jax
import jax.numpy as jnp
import numpy as np
from jax import lax
from jax.experimental import pallas as pl
from jax.experimental.pallas import tpu as pltpu
from jax.experimental.pallas import tpu_sc as plsc

N = 10000
E = 320000
DI = 128
DH = 128
DO = 64

NP = 10240
NC = 2
NS = 16
NW = NC * NS
B = 512
NB = 20
NB2 = 40
NCH = 2
NBC = NB2 // NCH
EP = NW * NB * B
RPS = NP // NS


@functools.lru_cache(maxsize=None)
def _mesh():
    return plsc.VectorSubcoreMesh(
        core_axis_name="c", subcore_axis_name="s", num_cores=NC, num_subcores=NS
    )


def _deg_body(init_hbm, dst_hbm, out_hbm, dst_v, ones_v, acc, sem):
    c = lax.axis_index("c")
    s = lax.axis_index("s")
    wid = s * NC + c
    pltpu.sync_copy(dst_hbm.at[wid], dst_v)
    for t in range(B // 16):
        ones_v[pl.ds(t * 16, 16)] = jnp.ones((16,), jnp.float32)
    pltpu.sync_copy(init_hbm.at[c, pl.ds(s * RPS, RPS)], acc.at[pl.ds(s * RPS, RPS)])
    plsc.subcore_barrier()

    def body(j, carry):
        pltpu.sync_copy(ones_v, acc.at[dst_v.at[j]], add=True)
        return carry

    lax.fori_loop(0, NB, body, 0)
    plsc.subcore_barrier()
    pltpu.sync_copy(acc.at[pl.ds(s * RPS, RPS)], out_hbm.at[c, pl.ds(s * RPS, RPS)])


@functools.lru_cache(maxsize=None)
def _deg_kernel():
    return pl.kernel(
        _deg_body,
        out_type=jax.ShapeDtypeStruct((NC, NP), jnp.float32),
        mesh=_mesh(),
        scratch_types=[
            pltpu.VMEM((NB, B), jnp.int32),
            pltpu.VMEM((B,), jnp.float32),
            pltpu.VMEM_SHARED((NP,), jnp.float32),
            pltpu.SemaphoreType.DMA,
        ],
        compiler_params=pltpu.CompilerParams(use_tc_tiling_on_sc=False),
    )


def _agg_body(ys_hbm, src0_hbm, dst_hbm, out_hbm,
              src_v, dst_v, rows0_v, rows1_v, acc, sg0, sg1):
    c = lax.axis_index("c")
    s = lax.axis_index("s")

    tbl = ys_hbm.at[pl.ds(c * NP, NP)]

    pltpu.sync_copy(ys_hbm.at[pl.ds(c * NP + s * RPS, RPS)],
                    acc.at[pl.ds(s * RPS, RPS)])
    plsc.subcore_barrier()

    def body(jj, carry):
        j = jj * 2
        d1 = pltpu.async_copy(tbl.at[src_v.at[j + 1]], rows1_v, sg1)
        pltpu.make_async_copy(tbl.at[src_v.at[0]], rows0_v, sg0).wait()
        pltpu.sync_copy(rows0_v, acc.at[dst_v.at[j]], add=True)

        @pl.when(jj + 1 < NBC // 2)
        def _():
            pltpu.async_copy(tbl.at[src_v.at[j + 2]], rows0_v, sg0)

        d1.wait()
        pltpu.sync_copy(rows1_v, acc.at[dst_v.at[j + 1]], add=True)
        return carry

    for ch in range(NCH):
        pltpu.sync_copy(src0_hbm.at[s, pl.ds(ch * NBC, NBC)], src_v)
        pltpu.sync_copy(dst_hbm.at[s, pl.ds(ch * NBC, NBC)], dst_v)
        pltpu.async_copy(tbl.at[src_v.at[0]], rows0_v, sg0)
        lax.fori_loop(0, NBC // 2, body, 0)

    plsc.subcore_barrier()
    pltpu.sync_copy(acc.at[pl.ds(s * RPS, RPS)], out_hbm.at[c, pl.ds(s * RPS, RPS)])


@functools.lru_cache(maxsize=None)
def _make_agg(d):
    return pl.kernel(
        _agg_body,
        out_type=jax.ShapeDtypeStruct((NC, NP, d), jnp.float32),
        mesh=_mesh(),
        scratch_types=[
            pltpu.VMEM((NBC, B), jnp.int32),
            pltpu.VMEM((NBC, B), jnp.int32),
            pltpu.VMEM((B, d), jnp.float32),
            pltpu.VMEM((B, d), jnp.float32),
            pltpu.VMEM_SHARED((NP, d), jnp.float32),
            pltpu.SemaphoreType.DMA,
            pltpu.SemaphoreType.DMA,
        ],
        compiler_params=pltpu.CompilerParams(use_tc_tiling_on_sc=False),
    )


def _tc_mm_body(x_ref, w0_ref, w1_ref, xw_ref):
    x = x_ref[...]
    xw_ref[0] = jnp.dot(x, w0_ref[...], preferred_element_type=jnp.float32)
    xw_ref[1] = jnp.dot(x, w1_ref[...], preferred_element_type=jnp.float32)


def _tc_pre_body(xw_ref, dp_ref, ys_ref, dis_ref):
    deg = dp_ref[0] + dp_ref[1]
    dis_c = jnp.where(deg > 0.0, lax.rsqrt(deg), 0.0).reshape(NP, 1)
    ys_ref[0] = xw_ref[0] * dis_c
    ys_ref[1] = xw_ref[1] * dis_c
    dis_ref[...] = dis_c


def _tc_mid_body(p_ref, dis_ref, b10_ref, b11_ref,
                 w00_ref, w01_ref, w10_ref, w11_ref, ys2_ref):
    dis = dis_ref[...]
    h0 = jnp.maximum(dis * p_ref[0] + b10_ref[...], 0.0)
    h1 = jnp.maximum(dis * p_ref[1] + b11_ref[...], 0.0)
    hw0 = (jnp.dot(h0, w00_ref[...], preferred_element_type=jnp.float32)
           + jnp.dot(h1, w10_ref[...], preferred_element_type=jnp.float32))
    hw1 = (jnp.dot(h0, w01_ref[...], preferred_element_type=jnp.float32)
           + jnp.dot(h1, w11_ref[...], preferred_element_type=jnp.float32))
    ys2_ref[0] = hw0 * dis
    ys2_ref[1] = hw1 * dis


def _tc_post_body(q_ref, dis_ref, b2_ref, out_ref):
    agg = jnp.concatenate([q_ref[0], q_ref[1]], axis=1)
    out_ref[...] = dis_ref[...] * agg + b2_ref[...]


def _tc_mm(x_pad, W1a, W1b):
    return pl.pallas_call(
        _tc_mm_body,
        in_specs=[
            pl.BlockSpec((NP, DI), lambda: (0, 0)),
            pl.BlockSpec((DI, DH // 2), lambda: (0, 0)),
            pl.BlockSpec((DI, DH // 2), lambda: (0, 0)),
        ],
        out_specs=pl.BlockSpec((NC, NP, DH // 2), lambda: (0, 0, 0)),
        out_shape=jax.ShapeDtypeStruct((NC, NP, DH // 2), jnp.float32),
    )(x_pad, W1a, W1b)


def _tc_pre(xw, degp):
    return pl.pallas_call(
        _tc_pre_body,
        in_specs=[
            pl.BlockSpec((NC, NP, DH // 2), lambda: (0, 0, 0)),
            pl.BlockSpec((NC, NP), lambda: (0, 0)),
        ],
        out_specs=[
            pl.BlockSpec((NC, NP, DH // 2), lambda: (0, 0, 0)),
            pl.BlockSpec((NP, 1), lambda: (0, 0)),
        ],
        out_shape=[
            jax.ShapeDtypeStruct((NC, NP, DH // 2), jnp.float32),
            jax.ShapeDtypeStruct((NP, 1), jnp.float32),
        ],
    )(xw, degp)


def _tc_mid(p, dis, b10, b11, W2s):
    return pl.pallas_call(
        _tc_mid_body,
        in_specs=[
            pl.BlockSpec((NC, NP, DH // 2), lambda: (0, 0, 0)),
            pl.BlockSpec((NP, 1), lambda: (0, 0)),
            pl.BlockSpec((1, DH // 2), lambda: (0, 0)),
            pl.BlockSpec((1, DH // 2), lambda: (0, 0)),
            pl.BlockSpec((DH // 2, DO // 2), lambda: (0, 0)),
            pl.BlockSpec((DH // 2, DO // 2), lambda: (0, 0)),
            pl.BlockSpec((DH // 2, DO // 2), lambda: (0, 0)),
            pl.BlockSpec((DH // 2, DO // 2), lambda: (0, 0)),
        ],
        out_specs=pl.BlockSpec((NC, NP, DO // 2), lambda: (0, 0, 0)),
        out_shape=jax.ShapeDtypeStruct((NC, NP, DO // 2), jnp.float32),
    )(p, dis, b10, b11, *W2s)


def _tc_post(q, dis, b2):
    R = 1000
    return pl.pallas_call(
        _tc_post_body,
        grid=(N // R,),
        in_specs=[
            pl.BlockSpec((NC, R, DO // 2), lambda i: (0, i, 0)),
            pl.BlockSpec((R, 1), lambda i: (i, 0)),
            pl.BlockSpec((1, DO), lambda i: (0, 0)),
        ],
        out_specs=pl.BlockSpec((R, DO), lambda i: (i, 0)),
        out_shape=jax.ShapeDtypeStruct((N, DO), jnp.float32),
    )(q, dis, b2)


_PAD_EDGES = np.stack([
    np.asarray((np.arange(EP - E) * 97) % N, np.int32),
    np.asarray(N + np.arange(EP - E) % (NP - N), np.int32),
])


def kernel(x, edge_index, W1, b1, W2, b2):
    ei = jnp.concatenate([edge_index.astype(jnp.int32),
                          jnp.asarray(_PAD_EDGES)], axis=1)
    dst_deg = ei[1].reshape(NW, NB, B)
    src_r0 = ei[0].reshape(NS, NB2, B)
    dst_r = ei[1].reshape(NS, NB2, B)

    x_pad = jnp.pad(x, ((0, NP - N), (0, 0)))
    deg_init = jnp.concatenate(
        [jnp.ones((1, NP), jnp.float32), jnp.zeros((1, NP), jnp.float32)]
    )
    h = DH // 2
    ho = DO // 2

    degp = _deg_kernel()(deg_init, dst_deg)
    xw = _tc_mm(x_pad, W1[:, :h], W1[:, h:])
    ys, dis = _tc_pre(xw, degp)
    ys_cat = ys.reshape(NC * NP, h)
    p = _make_agg(h)(ys_cat, src_r0, dst_r)
    W2s = (W2[:h, :ho], W2[:h, ho:], W2[h:, :ho], W2[h:, ho:])
    ys2 = _tc_mid(p, dis, b1.reshape(1, DH)[:, :h], b1.reshape(1, DH)[:, h:], W2s)
    ys2_cat = ys2.reshape(NC * NP, ho)
    q = _make_agg(ho)(ys2_cat, src_r0, dst_r)
    out = _tc_post(q, dis, b2.reshape(1, DO))
    return out
